# Initial kernel scaffold; baseline (speedup 1.0000x reference)
import jax
import jax.numpy as jnp
from jax.experimental import pallas as pl


def kernel(g, h, W_top, b_top, W_pool0, b_pool0, W_pool1, b_pool1, W_down0, b_down0, W_down1, b_down1):
    raise NotImplementedError("write your pallas kernel here")


# speedup vs baseline: 1.9703x; 1.9703x over previous
import functools

import jax
import jax.numpy as jnp
from jax import lax
from jax.experimental import pallas as pl
from jax.experimental.pallas import tpu as pltpu
from jax.experimental.pallas import tpu_sc as plsc

N = 4096
K0 = 392
K1 = 200
K0_PAD = 512
BM = 256
BI = 256
_F32 = jnp.float32
_BF16 = jnp.bfloat16


def _u_body(wt_ref, h_ref, u_ref):
    u_ref[...] = lax.dot_general(
        wt_ref[...], h_ref[...], (((1,), (1,)), ((), ())),
        preferred_element_type=_F32)


def _main_body(g_ref, u_ref, c_ref, h1_ref, s_ref):
    t = jnp.sum(g_ref[...] * u_ref[...], axis=1, keepdims=True)
    h1 = jnp.maximum(t + c_ref[0, 0], 0.0)
    s = jax.nn.sigmoid(h1 * c_ref[0, 1] + c_ref[0, 2])
    h1_ref[...] = h1
    s_ref[...] = s


def _rank_body(sc_ref, sr_ref, h1_ref, idx_ref, val_ref, v0_ref, p_ref):
    i = pl.program_id(0)

    @pl.when(i == 0)
    def _():
        idx_ref[...] = jnp.zeros_like(idx_ref)
        val_ref[...] = jnp.zeros_like(val_ref)
        v0_ref[...] = jnp.zeros_like(v0_ref)

    sc = sc_ref[...]
    sr = sr_ref[...]
    jglob = lax.broadcasted_iota(jnp.int32, (BI, N), 1)
    iglob = i * BI + lax.broadcasted_iota(jnp.int32, (BI, 1), 0)
    gt = (sr > sc).astype(jnp.int32)
    eq = ((sr == sc) & (jglob < iglob)).astype(jnp.int32)
    rank = (jnp.sum(gt, axis=1, keepdims=True)
            + jnp.sum(eq, axis=1, keepdims=True))
    m = lax.broadcasted_iota(jnp.int32, (BI, K0), 1)
    onehot = (rank == m).astype(_F32)
    p_ref[...] = onehot.astype(_BF16)
    dn = (((0,), (0,)), ((), ()))
    ifl = iglob.astype(_F32)
    idx_ref[...] += lax.dot_general(ifl, onehot, dn,
                                    preferred_element_type=_F32)
    val_ref[...] += lax.dot_general(sc, onehot, dn,
                                    preferred_element_type=_F32)
    v0_ref[...] += lax.dot_general(h1_ref[...] * sc, onehot, dn,
                                   preferred_element_type=_F32)


def _r2_body(g_ref, p_ref, r2_ref):
    rb = (g_ref[...] != 0.0).astype(_BF16)
    acc = lax.dot_general(rb, p_ref[...], (((1,), (0,)), ((), ())),
                          preferred_element_type=_F32)
    r2_ref[...] = acc.astype(_BF16)


def _gather_rows(g, idx_pad):
    bpw = K0_PAD // 32
    mesh = plsc.VectorSubcoreMesh(core_axis_name="c", subcore_axis_name="s")

    @functools.partial(
        pl.kernel,
        out_type=jax.ShapeDtypeStruct((K0_PAD, N), _F32),
        mesh=mesh,
        scratch_types=[
            pltpu.VMEM((bpw,), jnp.int32),
            pltpu.VMEM((bpw, N), _F32),
            pltpu.SemaphoreType.DMA,
        ],
    )
    def k(g_hbm, idx_hbm, out_hbm, idx_v, rows_v, sem):
        wid = lax.axis_index("s") * 2 + lax.axis_index("c")
        base = wid * bpw
        pltpu.sync_copy(idx_hbm.at[pl.ds(base, bpw)], idx_v)
        pltpu.async_copy(g_hbm.at[idx_v], rows_v, sem).wait()
        pltpu.sync_copy(rows_v, out_hbm.at[pl.ds(base, bpw)])

    return k(g, idx_pad)


def _tail_body(r1_ref, r2_ref, v0_ref, wd0_ref, wd1_ref, c_ref,
               h2_ref, h3_ref):
    dn_t = (((0,), (0,)), ((), ()))
    r1 = (r1_ref[0:K0, :] != 0.0).astype(_BF16)
    b = lax.dot_general(r1, r2_ref[...], (((1,), (0,)), ((), ())),
                        preferred_element_type=_F32)
    ung = (b != 0.0).astype(_F32)
    deg = jnp.sum(ung, axis=1, keepdims=True)
    g2 = ung / deg
    eye0 = (lax.broadcasted_iota(jnp.int32, (K0, K0), 0)
            == lax.broadcasted_iota(jnp.int32, (K0, K0), 1)).astype(_F32)
    w0c = lax.dot_general(eye0, v0_ref[...] * wd0_ref[...],
                          (((1,), (1,)), ((), ())),
                          preferred_element_type=_F32)
    h2 = jnp.maximum(
        lax.dot_general(g2, w0c, (((1,), (0,)), ((), ())),
                        preferred_element_type=_F32) + c_ref[0, 0], 0.0)
    s1 = jax.nn.sigmoid(h2 * c_ref[0, 1] + c_ref[0, 2])
    s1r = lax.dot_general(s1, eye0, dn_t,
                          preferred_element_type=_F32)
    im = lax.broadcasted_iota(jnp.int32, (K0, K0), 0)
    jm = lax.broadcasted_iota(jnp.int32, (K0, K0), 1)
    gt = (s1r > s1).astype(jnp.int32)
    eq = ((s1r == s1) & (jm < im)).astype(jnp.int32)
    rank2 = (jnp.sum(gt, axis=1, keepdims=True)
             + jnp.sum(eq, axis=1, keepdims=True))
    m2 = lax.broadcasted_iota(jnp.int32, (K0, K1), 1)
    o2 = (rank2 == m2).astype(_F32)
    vals1 = lax.dot_general(s1, o2, dn_t, preferred_element_type=_F32)
    h2sel = lax.dot_general(h2, o2, dn_t, preferred_element_type=_F32)
    v1 = vals1 * h2sel
    r2m = (g2 != 0.0).astype(_F32)
    t1 = lax.dot_general(o2, r2m, dn_t, preferred_element_type=_F32)
    t2 = lax.dot_general(r2m, o2, (((1,), (0,)), ((), ())),
                         preferred_element_type=_F32)
    b2 = lax.dot_general(t1, t2, (((1,), (0,)), ((), ())),
                         preferred_element_type=_F32)
    ung3 = (b2 != 0.0).astype(_F32)
    deg3 = jnp.sum(ung3, axis=1, keepdims=True)
    g3 = ung3 / deg3
    eye1 = (lax.broadcasted_iota(jnp.int32, (K1, K1), 0)
            == lax.broadcasted_iota(jnp.int32, (K1, K1), 1)).astype(_F32)
    w1c = lax.dot_general(eye1, v1 * wd1_ref[...],
                          (((1,), (1,)), ((), ())),
                          preferred_element_type=_F32)
    h3 = jnp.maximum(
        lax.dot_general(g3, w1c, (((1,), (0,)), ((), ())),
                        preferred_element_type=_F32) + c_ref[0, 3], 0.0)
    h2_ref[...] = h2
    h3_ref[...] = h3


def kernel(g, h, W_top, b_top, W_pool0, b_pool0, W_pool1, b_pool1,
           W_down0, b_down0, W_down1, b_down1):
    u_row = pl.pallas_call(
        _u_body,
        out_shape=jax.ShapeDtypeStruct((1, N), _F32))(W_top, h)

    c0 = jnp.concatenate([
        b_top.reshape(1, 1), W_pool0.reshape(1, 1), b_pool0.reshape(1, 1),
        jnp.zeros((1, 1), _F32)], axis=1)
    h1, s0 = pl.pallas_call(
        _main_body,
        grid=(N // BM,),
        in_specs=[
            pl.BlockSpec((BM, N), lambda i: (i, 0)),
            pl.BlockSpec((1, N), lambda i: (0, 0)),
            pl.BlockSpec((1, 4), lambda i: (0, 0)),
        ],
        out_specs=[
            pl.BlockSpec((BM, 1), lambda i: (i, 0)),
            pl.BlockSpec((BM, 1), lambda i: (i, 0)),
        ],
        out_shape=[
            jax.ShapeDtypeStruct((N, 1), _F32),
            jax.ShapeDtypeStruct((N, 1), _F32),
        ])(g, u_row, c0)

    s_row = s0.reshape(1, N)
    idxf, vals, v0, p = pl.pallas_call(
        _rank_body,
        grid=(N // BI,),
        in_specs=[
            pl.BlockSpec((BI, 1), lambda i: (i, 0)),
            pl.BlockSpec((1, N), lambda i: (0, 0)),
            pl.BlockSpec((BI, 1), lambda i: (i, 0)),
        ],
        out_specs=[
            pl.BlockSpec((1, K0), lambda i: (0, 0)),
            pl.BlockSpec((1, K0), lambda i: (0, 0)),
            pl.BlockSpec((1, K0), lambda i: (0, 0)),
            pl.BlockSpec((BI, K0), lambda i: (i, 0)),
        ],
        out_shape=[
            jax.ShapeDtypeStruct((1, K0), _F32),
            jax.ShapeDtypeStruct((1, K0), _F32),
            jax.ShapeDtypeStruct((1, K0), _F32),
            jax.ShapeDtypeStruct((N, K0), _BF16),
        ])(s0, s_row, h1)
    del vals

    r2 = pl.pallas_call(
        _r2_body,
        grid=(N // BM,),
        in_specs=[
            pl.BlockSpec((BM, N), lambda i: (i, 0)),
            pl.BlockSpec((N, K0), lambda i: (0, 0)),
        ],
        out_specs=pl.BlockSpec((BM, K0), lambda i: (i, 0)),
        out_shape=jax.ShapeDtypeStruct((N, K0), _BF16))(g, p)

    idx0 = idxf.reshape(K0).astype(jnp.int32)
    idx_pad = jnp.concatenate(
        [idx0, jnp.zeros((K0_PAD - K0,), jnp.int32)])
    r1 = _gather_rows(g, idx_pad)

    c1 = jnp.concatenate([
        b_down0.reshape(1, 1), W_pool1.reshape(1, 1), b_pool1.reshape(1, 1),
        b_down1.reshape(1, 1)], axis=1)
    h2, h3 = pl.pallas_call(
        _tail_body,
        out_shape=[
            jax.ShapeDtypeStruct((K0, 1), _F32),
            jax.ShapeDtypeStruct((K1, 1), _F32),
        ])(r1, r2, v0, W_down0, W_down1, c1)

    return jnp.concatenate([h1, h2, h3], axis=0)
